# baseline (speedup 1.0000x reference)
import functools

import jax
import jax.numpy as jnp
from jax import lax
from jax.experimental import pallas as pl
from jax.experimental.pallas import tpu as pltpu
from jax.experimental.pallas import tpu_sc as plsc

VOCAB = 1000000
EMBED = 32
HALF = EMBED // 2
B = 4
S = 4096

NUM_CORES = 2
NUM_SUBCORES = 16
NW = NUM_CORES * NUM_SUBCORES
CHUNK = (B * S) // NW
IDX_MINOR = 128
NDMA = CHUNK // IDX_MINOR
WORKERS_PER_B = NW // B
S_CHUNK = S // WORKERS_PER_B


def _sc_body(x_hbm, table_hbm, cos_hbm, sin_hbm, out_hbm,
             idx_v, rows_v, cos_v, sin_v, sem):
    cid = lax.axis_index("c")
    sid = lax.axis_index("s")
    wid = sid * NUM_CORES + cid
    b = wid // WORKERS_PER_B
    s0 = (wid % WORKERS_PER_B) * S_CHUNK

    pltpu.sync_copy(x_hbm.at[pl.ds(wid * NDMA, NDMA)], idx_v)
    copies = [
        pltpu.async_copy(
            table_hbm.at[idx_v.at[j]],
            rows_v.at[pl.ds(j * IDX_MINOR, IDX_MINOR)],
            sem,
        )
        for j in range(NDMA)
    ]
    pltpu.sync_copy(cos_hbm.at[pl.ds(s0, S_CHUNK)], cos_v)
    pltpu.sync_copy(sin_hbm.at[pl.ds(s0, S_CHUNK)], sin_v)
    for c in copies:
        c.wait()

    def row_body(k, carry):
        x1 = rows_v[k, pl.ds(0, HALF)]
        x2 = rows_v[k, pl.ds(HALF, HALF)]
        cv = cos_v[k]
        sv = sin_v[k]
        rows_v[k, pl.ds(0, HALF)] = x1 * cv - x2 * sv
        rows_v[k, pl.ds(HALF, HALF)] = x1 * sv + x2 * cv
        return carry

    lax.fori_loop(0, CHUNK, row_body, 0, unroll=4)

    pltpu.sync_copy(rows_v, out_hbm.at[pl.ds(s0, S_CHUNK), b])


@jax.jit
def kernel(x, token_embedding):
    xf = x.reshape(NW * NDMA, IDX_MINOR)

    theta = 1.0 / (10000.0 ** (jnp.arange(HALF, dtype=jnp.float32) / HALF))
    ang = jnp.arange(S, dtype=jnp.float32)[:, None] * theta[None, :]
    cos_t = jnp.cos(ang)
    sin_t = jnp.sin(ang)

    mesh = plsc.VectorSubcoreMesh(
        core_axis_name="c", subcore_axis_name="s",
        num_cores=NUM_CORES, num_subcores=NUM_SUBCORES,
    )
    run = pl.kernel(
        _sc_body,
        out_type=jax.ShapeDtypeStruct((S, B, EMBED), jnp.float32),
        mesh=mesh,
        scratch_types=[
            pltpu.VMEM((NDMA, IDX_MINOR), jnp.int32),
            pltpu.VMEM((CHUNK, EMBED), jnp.float32),
            pltpu.VMEM((S_CHUNK, HALF), jnp.float32),
            pltpu.VMEM((S_CHUNK, HALF), jnp.float32),
            pltpu.SemaphoreType.DMA,
        ],
        compiler_params=pltpu.CompilerParams(use_tc_tiling_on_sc=False),
    )
    return run(xf, token_embedding, cos_t, sin_t)
